# baseline (speedup 1.0000x reference)
import functools

import jax
import jax.numpy as jnp
from jax import lax
from jax.experimental import pallas as pl
from jax.experimental.pallas import tpu as pltpu
from jax.experimental.pallas import tpu_sc as plsc

N = 800000
S = 50000
D = 64
FC2 = 32
NW = 32
VW = 64
SEG_W = 784
S_PAD = SEG_W * VW
CH = 256
NBUF = 2


def _stage_start(idx_hbm, loc_hbm, seg_hbm, cb, idx_v, loc_v, seg_v, sem):
    pltpu.async_copy(idx_hbm.at[pl.ds(cb, CH)], idx_v, sem)
    pltpu.async_copy(loc_hbm.at[pl.ds(cb, CH)], loc_v, sem)
    pltpu.async_copy(seg_hbm.at[pl.ds(cb, CH)], seg_v, sem)


def _stage_drain(idx_hbm, loc_hbm, seg_hbm, cb, idx_v, loc_v, seg_v, sem):
    pltpu.make_async_copy(idx_hbm.at[pl.ds(cb, CH)], idx_v, sem).wait()
    pltpu.make_async_copy(loc_hbm.at[pl.ds(cb, CH)], loc_v, sem).wait()
    pltpu.make_async_copy(seg_hbm.at[pl.ds(cb, CH)], seg_v, sem).wait()


def _sc_body(idx_hbm, loc_hbm, seg_hbm, rs_hbm, gf_hbm, out_hbm,
             out_v, gath_v, loc_v, idx_v, seg_v, rs_v,
             sem_s0, sem_s1, sem_g0, sem_g1):
    wid = lax.axis_index("s") * 2 + lax.axis_index("c")
    sem_s = [sem_s0, sem_s1]
    sem_g = [sem_g0, sem_g1]

    neg = jnp.full((16,), -jnp.inf, jnp.float32)
    pltpu.sync_copy(rs_hbm, rs_v)

    def stage(cb, k):
        _stage_start(idx_hbm, loc_hbm, seg_hbm, cb,
                     idx_v.at[k], loc_v.at[k], seg_v.at[k], sem_s[k])

    def drain(cb, k):
        _stage_drain(idx_hbm, loc_hbm, seg_hbm, cb,
                     idx_v.at[k], loc_v.at[k], seg_v.at[k], sem_s[k])

    def gather_start(k):
        for h in range(CH // 128):
            pltpu.async_copy(gf_hbm.at[idx_v.at[k, pl.ds(h * 128, 128)]],
                             gath_v.at[k, pl.ds(h * 128, 128)], sem_g[k])

    def gather_wait(k):
        for h in range(CH // 128):
            pltpu.make_async_copy(
                gf_hbm.at[idx_v.at[k, pl.ds(h * 128, 128)]],
                gath_v.at[k, pl.ds(h * 128, 128)], sem_g[k]).wait()

    for p in range(2):
        v = wid * 2 + p
        seg_base = v * SEG_W

        def init_row(i, carry):
            for j in range(4):
                out_v[i, pl.ds(j * 16, 16)] = neg
            return carry

        lax.fori_loop(0, SEG_W + 2, init_row, 0)

        rs_vec = rs_v[pl.ds(v, 16)]
        rs = rs_vec[0]
        re = rs_vec[1]
        c0 = rs // CH
        c1 = (re + CH - 1) // CH
        nchunks = c1 - c0

        for b in range(NBUF):
            @pl.when(c0 + b < c1)
            def _():
                stage((c0 + b) * CH, b)

        @pl.when(nchunks > 0)
        def _():
            drain(c0 * CH, 0)
            gather_start(0)

        def group(g, carry):
            for k in range(NBUF):
                c = c0 + g * NBUF + k

                @pl.when(c < c1)
                def _():
                    cb = c * CH
                    kn1 = (k + 1) % NBUF

                    @pl.when(c + 1 < c1)
                    def _():
                        drain((c + 1) * CH, kn1)
                        gather_start(kn1)

                    gather_wait(k)

                    def block(bi, bcarry):
                        b16 = bi * 16
                        svec = seg_v[k, pl.ds(b16, 16)] - seg_base
                        svec = jnp.clip(svec, -1, SEG_W) + 1
                        for i in range(16):
                            sl = svec[i]
                            rl = b16 + i
                            for j in range(4):
                                g_ = (gath_v[k, rl, pl.ds(j * 16, 16)]
                                      + loc_v[k, rl, pl.ds(j * 16, 16)])
                                cur = out_v[sl, pl.ds(j * 16, 16)]
                                out_v[sl, pl.ds(j * 16, 16)] = \
                                    jnp.maximum(cur, g_)
                        return bcarry

                    lax.fori_loop(0, CH // 16, block, 0)

                    @pl.when(c + 2 < c1)
                    def _():
                        stage((c + 2) * CH, k)
            return carry

        lax.fori_loop(0, (nchunks + NBUF - 1) // NBUF, group, 0)

        pltpu.sync_copy(out_v.at[pl.ds(1, SEG_W)],
                        out_hbm.at[pl.ds(seg_base, SEG_W)])


_sc_pool = functools.partial(
    pl.kernel,
    mesh=plsc.VectorSubcoreMesh(core_axis_name="c", subcore_axis_name="s"),
    compiler_params=pltpu.CompilerParams(use_tc_tiling_on_sc=False),
    out_type=jax.ShapeDtypeStruct((S_PAD, D), jnp.float32),
    scratch_types=[
        pltpu.VMEM((SEG_W + 2, D), jnp.float32),
        pltpu.VMEM((NBUF, CH, D), jnp.float32),
        pltpu.VMEM((NBUF, CH, D), jnp.float32),
        pltpu.VMEM((NBUF, CH), jnp.int32),
        pltpu.VMEM((NBUF, CH), jnp.int32),
        pltpu.VMEM((80,), jnp.int32),
        pltpu.SemaphoreType.DMA,
        pltpu.SemaphoreType.DMA,
        pltpu.SemaphoreType.DMA,
        pltpu.SemaphoreType.DMA,
    ],
)(_sc_body)


def _mm_body(p_ref, w_ref, b_ref, o_ref):
    x = p_ref[...]
    x = jnp.where(x == -jnp.inf, jnp.float32(0.0), x)
    y = jnp.dot(x, w_ref[...], preferred_element_type=jnp.float32)
    o_ref[...] = jnp.maximum(y + b_ref[...], 0.0)


ROWS_BLK = 2000

_mm = pl.pallas_call(
    _mm_body,
    grid=(S // ROWS_BLK,),
    in_specs=[
        pl.BlockSpec((ROWS_BLK, D), lambda i: (i, 0)),
        pl.BlockSpec((D, FC2), lambda i: (0, 0)),
        pl.BlockSpec((1, FC2), lambda i: (0, 0)),
    ],
    out_specs=pl.BlockSpec((ROWS_BLK, FC2), lambda i: (i, 0)),
    out_shape=jax.ShapeDtypeStruct((S, FC2), jnp.float32),
)


def kernel(shuffled_ind, pt_gfeatures, pt_localfeatures2, pt_inv, W, b):
    idx = shuffled_ind.astype(jnp.int32)
    seg = pt_inv.astype(jnp.int32)
    bounds = jnp.arange(VW + 1, dtype=jnp.int32) * SEG_W
    rs = jnp.searchsorted(seg, bounds, side="left").astype(jnp.int32)
    rs = jnp.concatenate([rs, jnp.zeros((15,), jnp.int32)])
    pooled = _sc_pool(idx, pt_localfeatures2, seg, rs, pt_gfeatures)
    return _mm(pooled[:S], W, b.reshape(1, FC2))


# speedup vs baseline: 1.0775x; 1.0775x over previous
import functools

import jax
import jax.numpy as jnp
from jax import lax
from jax.experimental import pallas as pl
from jax.experimental.pallas import tpu as pltpu
from jax.experimental.pallas import tpu_sc as plsc

N = 800000
S = 50000
D = 64
FC2 = 32
NW = 32
VW = 64
SEG_W = 784
S_PAD = SEG_W * VW
CH = 128
NBUF = 3


def _stage_start(idx_hbm, loc_hbm, seg_hbm, cb, idx_v, loc_v, seg_v, sem):
    pltpu.async_copy(idx_hbm.at[pl.ds(cb, CH)], idx_v, sem)
    pltpu.async_copy(loc_hbm.at[pl.ds(cb, CH)], loc_v, sem)
    pltpu.async_copy(seg_hbm.at[pl.ds(cb, CH)], seg_v, sem)


def _stage_drain(idx_hbm, loc_hbm, seg_hbm, cb, idx_v, loc_v, seg_v, sem):
    pltpu.make_async_copy(idx_hbm.at[pl.ds(cb, CH)], idx_v, sem).wait()
    pltpu.make_async_copy(loc_hbm.at[pl.ds(cb, CH)], loc_v, sem).wait()
    pltpu.make_async_copy(seg_hbm.at[pl.ds(cb, CH)], seg_v, sem).wait()


def _sc_body(idx_hbm, loc_hbm, seg_hbm, rs_hbm, gf_hbm, out_hbm,
             out_v, gath_v, loc_v, idx_v, seg_v, rs_v,
             sem_s0, sem_s1, sem_s2, sem_g0, sem_g1, sem_g2):
    wid = lax.axis_index("s") * 2 + lax.axis_index("c")
    sem_s = [sem_s0, sem_s1, sem_s2]
    sem_g = [sem_g0, sem_g1, sem_g2]

    neg = jnp.full((16,), -jnp.inf, jnp.float32)
    pltpu.sync_copy(rs_hbm, rs_v)

    def stage(cb, k):
        _stage_start(idx_hbm, loc_hbm, seg_hbm, cb,
                     idx_v.at[k], loc_v.at[k], seg_v.at[k], sem_s[k])

    def drain(cb, k):
        _stage_drain(idx_hbm, loc_hbm, seg_hbm, cb,
                     idx_v.at[k], loc_v.at[k], seg_v.at[k], sem_s[k])

    def gather_start(k):
        for h in range(CH // 128):
            pltpu.async_copy(gf_hbm.at[idx_v.at[k, pl.ds(h * 128, 128)]],
                             gath_v.at[k, pl.ds(h * 128, 128)], sem_g[k])

    def gather_wait(k):
        for h in range(CH // 128):
            pltpu.make_async_copy(
                gf_hbm.at[idx_v.at[k, pl.ds(h * 128, 128)]],
                gath_v.at[k, pl.ds(h * 128, 128)], sem_g[k]).wait()

    for p in range(2):
        v = wid * 2 + p
        seg_base = v * SEG_W

        def init_row(i, carry):
            for j in range(4):
                out_v[i, pl.ds(j * 16, 16)] = neg
            return carry

        lax.fori_loop(0, SEG_W + 2, init_row, 0)

        rs_vec = rs_v[pl.ds(v, 16)]
        rs = rs_vec[0]
        re = rs_vec[1]
        c0 = rs // CH
        c1 = (re + CH - 1) // CH
        nchunks = c1 - c0

        for b in range(NBUF - 1):
            @pl.when(c0 + b < c1)
            def _():
                stage((c0 + b) * CH, b)

        @pl.when(nchunks > 0)
        def _():
            drain(c0 * CH, 0)
            gather_start(0)

        def group(g, carry):
            for k in range(NBUF):
                c = c0 + g * NBUF + k

                @pl.when(c < c1)
                def _():
                    cb = c * CH
                    kn1 = (k + 1) % NBUF
                    kn2 = (k + 2) % NBUF

                    @pl.when(c + 1 < c1)
                    def _():
                        drain((c + 1) * CH, kn1)
                        gather_start(kn1)

                    @pl.when(c + 2 < c1)
                    def _():
                        stage((c + 2) * CH, kn2)

                    gather_wait(k)

                    def block(bi, bcarry):
                        b16 = bi * 16
                        svec = seg_v[k, pl.ds(b16, 16)] - seg_base
                        svec = jnp.clip(svec, -1, SEG_W) + 1
                        for i in range(16):
                            sl = svec[i]
                            rl = b16 + i
                            for j in range(4):
                                g_ = (gath_v[k, rl, pl.ds(j * 16, 16)]
                                      + loc_v[k, rl, pl.ds(j * 16, 16)])
                                cur = out_v[sl, pl.ds(j * 16, 16)]
                                out_v[sl, pl.ds(j * 16, 16)] = \
                                    jnp.maximum(cur, g_)
                        return bcarry

                    lax.fori_loop(0, CH // 16, block, 0)
            return carry

        lax.fori_loop(0, (nchunks + NBUF - 1) // NBUF, group, 0)

        pltpu.sync_copy(out_v.at[pl.ds(1, SEG_W)],
                        out_hbm.at[pl.ds(seg_base, SEG_W)])


_sc_pool = functools.partial(
    pl.kernel,
    mesh=plsc.VectorSubcoreMesh(core_axis_name="c", subcore_axis_name="s"),
    compiler_params=pltpu.CompilerParams(use_tc_tiling_on_sc=False),
    out_type=jax.ShapeDtypeStruct((S_PAD, D), jnp.float32),
    scratch_types=[
        pltpu.VMEM((SEG_W + 2, D), jnp.float32),
        pltpu.VMEM((NBUF, CH, D), jnp.float32),
        pltpu.VMEM((NBUF, CH, D), jnp.float32),
        pltpu.VMEM((NBUF, CH), jnp.int32),
        pltpu.VMEM((NBUF, CH), jnp.int32),
        pltpu.VMEM((80,), jnp.int32),
        pltpu.SemaphoreType.DMA,
        pltpu.SemaphoreType.DMA,
        pltpu.SemaphoreType.DMA,
        pltpu.SemaphoreType.DMA,
        pltpu.SemaphoreType.DMA,
        pltpu.SemaphoreType.DMA,
    ],
)(_sc_body)


def _mm_body(p_ref, w_ref, b_ref, o_ref):
    x = p_ref[...]
    x = jnp.where(x == -jnp.inf, jnp.float32(0.0), x)
    y = jnp.dot(x, w_ref[...], preferred_element_type=jnp.float32)
    o_ref[...] = jnp.maximum(y + b_ref[...], 0.0)


ROWS_BLK = 2000

_mm = pl.pallas_call(
    _mm_body,
    grid=(S // ROWS_BLK,),
    in_specs=[
        pl.BlockSpec((ROWS_BLK, D), lambda i: (i, 0)),
        pl.BlockSpec((D, FC2), lambda i: (0, 0)),
        pl.BlockSpec((1, FC2), lambda i: (0, 0)),
    ],
    out_specs=pl.BlockSpec((ROWS_BLK, FC2), lambda i: (i, 0)),
    out_shape=jax.ShapeDtypeStruct((S, FC2), jnp.float32),
)


def kernel(shuffled_ind, pt_gfeatures, pt_localfeatures2, pt_inv, W, b):
    idx = shuffled_ind.astype(jnp.int32)
    seg = pt_inv.astype(jnp.int32)
    bounds = jnp.arange(VW + 1, dtype=jnp.int32) * SEG_W
    rs = jnp.searchsorted(seg, bounds, side="left").astype(jnp.int32)
    rs = jnp.concatenate([rs, jnp.zeros((15,), jnp.int32)])
    pooled = _sc_pool(idx, pt_localfeatures2, seg, rs, pt_gfeatures)
    return _mm(pooled[:S], W, b.reshape(1, FC2))
